# baseline (speedup 1.0000x reference)
import jax
import jax.numpy as jnp
from jax import lax
from jax.experimental import pallas as pl
from jax.experimental.pallas import tpu as pltpu
from jax.experimental.pallas import tpu_sc as plsc

DIM = 32
LANES = 16
NC, NS = 2, 16
NW = NC * NS
BATCH = 16384
BPW = BATCH // NW
GROUPS = BPW // LANES
CHUNK = 128
NCHUNK = BPW // CHUNK


def _sc_body(data_hbm, user_hbm, item_hbm, ent_hbm, out_hbm,
             data_v, uidx_v, pidx_v, nidx_v,
             urows, pirows, perows, nirows, nerows, scores_v, sem):
    wid = lax.axis_index("c") * NS + lax.axis_index("s")
    base = wid * BPW

    pltpu.sync_copy(data_hbm.at[pl.ds(base, BPW), :], data_v)

    iota = lax.iota(jnp.int32, LANES)
    col0 = jnp.zeros((LANES,), jnp.int32)
    col1 = col0 + 1
    col2 = col0 + 2

    def split_body(g, carry):
        rows = g * LANES + iota
        uidx_v[pl.ds(g * LANES, LANES)] = plsc.load_gather(data_v, [rows, col0])
        pidx_v[pl.ds(g * LANES, LANES)] = plsc.load_gather(data_v, [rows, col1])
        nidx_v[pl.ds(g * LANES, LANES)] = plsc.load_gather(data_v, [rows, col2])
        return carry
    lax.fori_loop(0, GROUPS, split_body, 0)

    copies = []
    for k in range(NCHUNK):
        sl = pl.ds(k * CHUNK, CHUNK)
        copies.append(pltpu.async_copy(
            user_hbm.at[uidx_v.at[sl]], urows.at[sl, :], sem))
        copies.append(pltpu.async_copy(
            item_hbm.at[pidx_v.at[sl]], pirows.at[sl, :], sem))
        copies.append(pltpu.async_copy(
            ent_hbm.at[pidx_v.at[sl]], perows.at[sl, :], sem))
        copies.append(pltpu.async_copy(
            item_hbm.at[nidx_v.at[sl]], nirows.at[sl, :], sem))
        copies.append(pltpu.async_copy(
            ent_hbm.at[nidx_v.at[sl]], nerows.at[sl, :], sem))
    for c in copies:
        c.wait()

    def dot_body(g, carry):
        rows = g * LANES + iota
        acc = jnp.zeros((LANES,), jnp.float32)
        for d in range(DIM):
            cold = jnp.full((LANES,), d, jnp.int32)
            uv = plsc.load_gather(urows, [rows, cold])
            pv = plsc.load_gather(pirows, [rows, cold]) + plsc.load_gather(perows, [rows, cold])
            nv = plsc.load_gather(nirows, [rows, cold]) + plsc.load_gather(nerows, [rows, cold])
            acc = acc + uv * (pv - nv)
        scores_v[pl.ds(g * LANES, LANES)] = acc
        return carry
    lax.fori_loop(0, GROUPS, dot_body, 0)

    pltpu.sync_copy(scores_v, out_hbm.at[pl.ds(base, BPW)])


_sc_diff = pl.kernel(
    _sc_body,
    out_type=jax.ShapeDtypeStruct((BATCH,), jnp.float32),
    mesh=plsc.VectorSubcoreMesh(core_axis_name="c", subcore_axis_name="s"),
    compiler_params=pltpu.CompilerParams(
        needs_layout_passes=False, use_tc_tiling_on_sc=False),
    scratch_types=[
        pltpu.VMEM((BPW, 3), jnp.int32),
        pltpu.VMEM((BPW,), jnp.int32),
        pltpu.VMEM((BPW,), jnp.int32),
        pltpu.VMEM((BPW,), jnp.int32),
        pltpu.VMEM((BPW, DIM), jnp.float32),
        pltpu.VMEM((BPW, DIM), jnp.float32),
        pltpu.VMEM((BPW, DIM), jnp.float32),
        pltpu.VMEM((BPW, DIM), jnp.float32),
        pltpu.VMEM((BPW, DIM), jnp.float32),
        pltpu.VMEM((BPW,), jnp.float32),
        pltpu.SemaphoreType.DMA,
    ],
)


def _tc_body(x_ref, o_ref):
    x = x_ref[...]
    y = jnp.minimum(x, 0.0) - jnp.log1p(jnp.exp(-jnp.abs(x)))
    o_ref[0, 0] = jnp.sum(y)


_tc_logsig_sum = pl.pallas_call(
    _tc_body,
    out_shape=jax.ShapeDtypeStruct((1, 1), jnp.float32),
    in_specs=[pl.BlockSpec(memory_space=pltpu.VMEM)],
    out_specs=pl.BlockSpec(memory_space=pltpu.SMEM),
)


def kernel(data, name, user_emb_matrix, item_emb_matrix, ent_emb_matrix, Mr_matrix, rel_emb_matrix):
    del name, Mr_matrix, rel_emb_matrix
    diff = _sc_diff(data, user_emb_matrix, item_emb_matrix, ent_emb_matrix)
    total = _tc_logsig_sum(diff.reshape(BATCH // 128, 128))
    return total[0, 0]


# speedup vs baseline: 1.0036x; 1.0036x over previous
import jax
import jax.numpy as jnp
from jax import lax
from jax.experimental import pallas as pl
from jax.experimental.pallas import tpu as pltpu
from jax.experimental.pallas import tpu_sc as plsc

DIM = 32
LANES = 16
NC, NS = 2, 16
NW = NC * NS
BATCH = 16384
BPW = BATCH // NW
ROWPACK = 128 // DIM
CHUNK = 128
NCHUNK = BPW // CHUNK
CGROUPS = CHUNK // LANES


def _sc_body(uidx_hbm, pidx_hbm, nidx_hbm, user_hbm, item_hbm, ent_hbm,
             out_hbm,
             uidx_v, pidx_v, nidx_v, ublk_v, pblk_v, nblk_v,
             urows, pirows, perows, nirows, nerows, scores_v, sem):
    wid = lax.axis_index("c") * NS + lax.axis_index("s")
    base = wid * BPW

    pltpu.sync_copy(uidx_hbm.at[pl.ds(base, BPW)], uidx_v)
    pltpu.sync_copy(pidx_hbm.at[pl.ds(base, BPW)], pidx_v)
    pltpu.sync_copy(nidx_hbm.at[pl.ds(base, BPW)], nidx_v)

    iota = lax.iota(jnp.int32, LANES)

    def blk_body(g, carry):
        sl = pl.ds(g * LANES, LANES)
        ublk_v[sl] = lax.shift_right_logical(uidx_v[sl], 2)
        pblk_v[sl] = lax.shift_right_logical(pidx_v[sl], 2)
        nblk_v[sl] = lax.shift_right_logical(nidx_v[sl], 2)
        return carry
    lax.fori_loop(0, BPW // LANES, blk_body, 0)

    for ch in range(NCHUNK):
        sl = pl.ds(ch * CHUNK, CHUNK)
        copies = [
            pltpu.async_copy(user_hbm.at[ublk_v.at[sl]], urows, sem),
            pltpu.async_copy(item_hbm.at[pblk_v.at[sl]], pirows, sem),
            pltpu.async_copy(ent_hbm.at[pblk_v.at[sl]], perows, sem),
            pltpu.async_copy(item_hbm.at[nblk_v.at[sl]], nirows, sem),
            pltpu.async_copy(ent_hbm.at[nblk_v.at[sl]], nerows, sem),
        ]
        for c in copies:
            c.wait()

        def dot_body(g, carry, ch=ch):
            rows = g * LANES + iota
            gsl = pl.ds(ch * CHUNK + g * LANES, LANES)
            uoff = (uidx_v[gsl] & 3) * DIM
            poff = (pidx_v[gsl] & 3) * DIM
            noff = (nidx_v[gsl] & 3) * DIM
            acc = jnp.zeros((LANES,), jnp.float32)
            for d in range(DIM):
                uv = plsc.load_gather(urows, [rows, uoff + d])
                pv = (plsc.load_gather(pirows, [rows, poff + d])
                      + plsc.load_gather(perows, [rows, poff + d]))
                nv = (plsc.load_gather(nirows, [rows, noff + d])
                      + plsc.load_gather(nerows, [rows, noff + d]))
                acc = acc + uv * (pv - nv)
            scores_v[gsl] = acc
            return carry
        lax.fori_loop(0, CGROUPS, dot_body, 0)

    pltpu.sync_copy(scores_v, out_hbm.at[pl.ds(base, BPW)])


_sc_diff = pl.kernel(
    _sc_body,
    out_type=jax.ShapeDtypeStruct((BATCH,), jnp.float32),
    mesh=plsc.VectorSubcoreMesh(core_axis_name="c", subcore_axis_name="s"),
    compiler_params=pltpu.CompilerParams(needs_layout_passes=False),
    scratch_types=[
        pltpu.VMEM((BPW,), jnp.int32),
        pltpu.VMEM((BPW,), jnp.int32),
        pltpu.VMEM((BPW,), jnp.int32),
        pltpu.VMEM((BPW,), jnp.int32),
        pltpu.VMEM((BPW,), jnp.int32),
        pltpu.VMEM((BPW,), jnp.int32),
        pltpu.VMEM((CHUNK, 4 * DIM), jnp.float32),
        pltpu.VMEM((CHUNK, 4 * DIM), jnp.float32),
        pltpu.VMEM((CHUNK, 4 * DIM), jnp.float32),
        pltpu.VMEM((CHUNK, 4 * DIM), jnp.float32),
        pltpu.VMEM((CHUNK, 4 * DIM), jnp.float32),
        pltpu.VMEM((BPW,), jnp.float32),
        pltpu.SemaphoreType.DMA,
    ],
)


def _tc_body(x_ref, o_ref):
    x = x_ref[...]
    y = jnp.minimum(x, 0.0) - jnp.log1p(jnp.exp(-jnp.abs(x)))
    o_ref[0, 0] = jnp.sum(y)


_tc_logsig_sum = pl.pallas_call(
    _tc_body,
    out_shape=jax.ShapeDtypeStruct((1, 1), jnp.float32),
    in_specs=[pl.BlockSpec(memory_space=pltpu.VMEM)],
    out_specs=pl.BlockSpec(memory_space=pltpu.SMEM),
)


def kernel(data, name, user_emb_matrix, item_emb_matrix, ent_emb_matrix, Mr_matrix, rel_emb_matrix):
    del name, Mr_matrix, rel_emb_matrix
    n_packed = user_emb_matrix.shape[0] // ROWPACK
    t_user = user_emb_matrix.reshape(n_packed, ROWPACK * DIM)
    t_item = item_emb_matrix.reshape(n_packed, ROWPACK * DIM)
    t_ent = ent_emb_matrix.reshape(n_packed, ROWPACK * DIM)
    diff = _sc_diff(data[:, 0], data[:, 1], data[:, 2], t_user, t_item, t_ent)
    total = _tc_logsig_sum(diff.reshape(BATCH // 128, 128))
    return total[0, 0]
